# baseline (device time: 76406 ns/iter reference)
import jax
import jax.numpy as jnp
from jax import lax
from jax.experimental import pallas as pl
from jax.experimental.pallas import tpu as pltpu

N_DEV = 4
SCALE = 0.08838834764831843
SQ = 256
SKV = 4096
HQ = 8
DH = 128
NB = 4
BQ = 64
GK = SKV // (NB * BQ)
KV_R = GK * BQ
D_MODEL = HQ * DH


def kernel(x, Wq, K_ext, V_ext, Wo):
    x2 = x.reshape(SQ, D_MODEL)
    K4 = K_ext.reshape(GK, NB, BQ, D_MODEL).astype(jnp.bfloat16)
    V4 = V_ext.reshape(GK, NB, BQ, D_MODEL).astype(jnp.bfloat16)

    def body(x_ref, wq_ref, k_ref, v_ref, wo_ref, out_ref,
             qbuf, psrc, pbuf, lsrc, lbuf, pme_o, pme_l,
             kp_ref, vp_ref,
             qs_sems, qr_sems, ps_sems, pr_sems, ls_sems, lr_sems,
             kv_sems):
        my = lax.axis_index("i")
        left = (my + N_DEV - 1) % N_DEV
        right = (my + 1) % N_DEV
        diag = (my + 2) % N_DEV

        kv_copies = []
        for r in range(NB):
            c = pltpu.make_async_copy(k_ref.at[:, r], kp_ref.at[r],
                                      kv_sems.at[r])
            c.start()
            kv_copies.append(c)
            c = pltpu.make_async_copy(v_ref.at[:, r], vp_ref.at[r],
                                      kv_sems.at[NB + r])
            c.start()
            kv_copies.append(c)

        barrier_sem = pltpu.get_barrier_semaphore()
        for nbr in (left, right, diag):
            pl.semaphore_signal(barrier_sem, inc=1, device_id=(nbr,),
                                device_id_type=pl.DeviceIdType.MESH)

        targets = (right, diag, left)

        def direct(src, dst, ssems, rsems, k):
            return pltpu.make_async_remote_copy(
                src_ref=src, dst_ref=dst,
                send_sem=ssems.at[k], recv_sem=rsems.at[k],
                device_id=(targets[k],),
                device_id_type=pl.DeviceIdType.MESH)

        qsend = [direct(qbuf.at[0], qbuf.at[k + 1], qs_sems, qr_sems, k)
                 for k in range(3)]
        psend = [direct(psrc.at[k], pbuf.at[k], ps_sems, pr_sems, k)
                 for k in range(3)]
        lsend = [direct(lsrc.at[k], lbuf.at[k], ls_sems, lr_sems, k)
                 for k in range(3)]

        q = jnp.dot(x_ref[...].astype(jnp.bfloat16),
                    wq_ref[...].astype(jnp.bfloat16),
                    preferred_element_type=jnp.float32) * SCALE
        pl.semaphore_wait(barrier_sem, 3)
        qbuf[0, :, :] = q.astype(jnp.bfloat16)
        for k in range(3):
            qsend[k].start()

        for c in kv_copies:
            c.wait()

        def partial2(slot_a, slot_b, oa, la, ob, lb):
            for h in range(HQ):
                hc = slice(h * DH, (h + 1) * DH)
                qv = jnp.concatenate(
                    [qbuf[slot_a, :, hc].reshape(NB, BQ, DH),
                     qbuf[slot_b, :, hc].reshape(NB, BQ, DH)],
                    axis=1)
                ks = kp_ref[:, :, :, hc].reshape(NB, KV_R, DH)
                vs = vp_ref[:, :, :, hc].reshape(NB, KV_R, DH)
                s = lax.dot_general(
                    qv, ks, (((2,), (2,)), ((0,), (0,))),
                    preferred_element_type=jnp.float32)
                e = jnp.exp(s)
                o = lax.dot_general(
                    e.astype(jnp.bfloat16), vs, (((2,), (1,)), ((0,), (0,))),
                    preferred_element_type=jnp.float32)
                lsum = jnp.sum(e, axis=2)
                oa[:, hc] = o[:, :BQ, :].reshape(SQ, DH).astype(oa.dtype)
                ob[:, hc] = o[:, BQ:, :].reshape(SQ, DH).astype(ob.dtype)
                la[:, h:h + 1] = lsum[:, :BQ].reshape(SQ, 1)
                lb[:, h:h + 1] = lsum[:, BQ:].reshape(SQ, 1)

        qsend[2].wait_recv()
        partial2(0, 3, pme_o, pme_l, psrc.at[0], lsrc.at[0])
        psend[0].start()
        lsend[0].start()

        qsend[1].wait_recv()
        qsend[0].wait_recv()
        partial2(2, 1, psrc.at[1], lsrc.at[1], psrc.at[2], lsrc.at[2])
        psend[1].start()
        lsend[1].start()
        psend[2].start()
        lsend[2].start()

        psend[0].wait_recv()
        lsend[0].wait_recv()
        psend[1].wait_recv()
        lsend[1].wait_recv()
        o01 = (pme_o[:, :] + pbuf[0, :, :].astype(jnp.float32)
               + pbuf[1, :, :].astype(jnp.float32))
        l01 = pme_l[:, :] + lbuf[0, :, :] + lbuf[1, :, :]
        psend[2].wait_recv()
        lsend[2].wait_recv()
        o_sum = o01 + pbuf[2, :, :].astype(jnp.float32)
        l_sum = l01 + lbuf[2, :, :]
        ctx = jnp.concatenate(
            [o_sum[:, h * DH:(h + 1) * DH] / l_sum[:, h:h + 1]
             for h in range(HQ)], axis=1)
        out_ref[...] = jnp.dot(ctx.astype(jnp.bfloat16),
                               wo_ref[...].astype(jnp.bfloat16),
                               preferred_element_type=jnp.float32)

        for k in range(3):
            qsend[k].wait_send()
            psend[k].wait_send()
            lsend[k].wait_send()

    out = pl.pallas_call(
        body,
        out_shape=jax.ShapeDtypeStruct((SQ, D_MODEL), jnp.float32),
        in_specs=[
            pl.BlockSpec(memory_space=pltpu.VMEM),
            pl.BlockSpec(memory_space=pltpu.VMEM),
            pl.BlockSpec(memory_space=pl.ANY),
            pl.BlockSpec(memory_space=pl.ANY),
            pl.BlockSpec(memory_space=pltpu.VMEM),
        ],
        out_specs=pl.BlockSpec(memory_space=pltpu.VMEM),
        scratch_shapes=[
            pltpu.VMEM((N_DEV, SQ, D_MODEL), jnp.bfloat16),
            pltpu.VMEM((3, SQ, D_MODEL), jnp.bfloat16),
            pltpu.VMEM((3, SQ, D_MODEL), jnp.bfloat16),
            pltpu.VMEM((3, SQ, HQ), jnp.float32),
            pltpu.VMEM((3, SQ, HQ), jnp.float32),
            pltpu.VMEM((SQ, D_MODEL), jnp.float32),
            pltpu.VMEM((SQ, HQ), jnp.float32),
            pltpu.VMEM((NB, GK, BQ, D_MODEL), jnp.bfloat16),
            pltpu.VMEM((NB, GK, BQ, D_MODEL), jnp.bfloat16),
            pltpu.SemaphoreType.DMA((3,)),
            pltpu.SemaphoreType.DMA((3,)),
            pltpu.SemaphoreType.DMA((3,)),
            pltpu.SemaphoreType.DMA((3,)),
            pltpu.SemaphoreType.DMA((3,)),
            pltpu.SemaphoreType.DMA((3,)),
            pltpu.SemaphoreType.DMA((2 * NB,)),
        ],
        compiler_params=pltpu.CompilerParams(
            collective_id=0, vmem_limit_bytes=60 * 1024 * 1024),
    )(x2, Wq, K4, V4, Wo)
    return out.reshape(1, SQ, D_MODEL)


# device time: 72373 ns/iter; 1.0557x vs baseline; 1.0557x over previous
import jax
import jax.numpy as jnp
from jax import lax
from jax.experimental import pallas as pl
from jax.experimental.pallas import tpu as pltpu

N_DEV = 4
SCALE = 0.08838834764831843
SQ = 256
SKV = 4096
HQ = 8
DH = 128
NB = 4
BQ = 64
GK = SKV // (NB * BQ)
KV_R = GK * BQ
D_MODEL = HQ * DH


def kernel(x, Wq, K_ext, V_ext, Wo):
    x2 = x.reshape(SQ, D_MODEL)
    K4 = K_ext.reshape(GK, NB, BQ, D_MODEL).astype(jnp.bfloat16)
    V4 = V_ext.reshape(GK, NB, BQ, D_MODEL).astype(jnp.bfloat16)

    def body(x_ref, wq_ref, k_ref, v_ref, wo_ref, out_ref,
             qbuf, psrc, pbuf, lsrc, lbuf, pme_o, pme_l,
             kp_ref, vp_ref,
             qs_sems, qr_sems, ps_sems, pr_sems, ls_sems, lr_sems,
             kv_sems):
        my = lax.axis_index("i")
        left = (my + N_DEV - 1) % N_DEV
        right = (my + 1) % N_DEV
        diag = (my + 2) % N_DEV

        kv_copies = []
        for r in range(NB):
            c = pltpu.make_async_copy(k_ref.at[:, r], kp_ref.at[r],
                                      kv_sems.at[r])
            c.start()
            kv_copies.append(c)
            c = pltpu.make_async_copy(v_ref.at[:, r], vp_ref.at[r],
                                      kv_sems.at[NB + r])
            c.start()
            kv_copies.append(c)

        barrier_sem = pltpu.get_barrier_semaphore()
        for nbr in (left, right, diag):
            pl.semaphore_signal(barrier_sem, inc=1, device_id=(nbr,),
                                device_id_type=pl.DeviceIdType.MESH)

        targets = (right, diag, left)

        def direct(src, dst, ssems, rsems, k):
            return pltpu.make_async_remote_copy(
                src_ref=src, dst_ref=dst,
                send_sem=ssems.at[k], recv_sem=rsems.at[k],
                device_id=(targets[k],),
                device_id_type=pl.DeviceIdType.MESH)

        qsend = [direct(qbuf.at[0], qbuf.at[k + 1], qs_sems, qr_sems, k)
                 for k in range(3)]
        psend = [direct(psrc.at[k], pbuf.at[k], ps_sems, pr_sems, k)
                 for k in range(2)]
        lsend = [direct(lsrc.at[k], lbuf.at[k], ls_sems, lr_sems, k)
                 for k in range(3)]
        DH2 = D_MODEL // 2
        psend2 = [pltpu.make_async_remote_copy(
                      src_ref=psrc.at[2, :, c0:c1],
                      dst_ref=pbuf.at[2, :, c0:c1],
                      send_sem=ps_sems.at[2 + i], recv_sem=pr_sems.at[2 + i],
                      device_id=(left,), device_id_type=pl.DeviceIdType.MESH)
                  for i, (c0, c1) in enumerate(((0, DH2), (DH2, D_MODEL)))]

        q = jnp.dot(x_ref[...].astype(jnp.bfloat16),
                    wq_ref[...].astype(jnp.bfloat16),
                    preferred_element_type=jnp.float32) * SCALE
        pl.semaphore_wait(barrier_sem, 3)
        qbuf[0, :, :] = q.astype(jnp.bfloat16)
        for k in range(3):
            qsend[k].start()

        for c in kv_copies:
            c.wait()

        def partial2(slot_a, slot_b, oa, la, ob, lb, mid=None):
            for h in range(HQ):
                if h == HQ // 2 and mid is not None:
                    mid()
                hc = slice(h * DH, (h + 1) * DH)
                qv = jnp.concatenate(
                    [qbuf[slot_a, :, hc].reshape(NB, BQ, DH),
                     qbuf[slot_b, :, hc].reshape(NB, BQ, DH)],
                    axis=1)
                ks = kp_ref[:, :, :, hc].reshape(NB, KV_R, DH)
                vs = vp_ref[:, :, :, hc].reshape(NB, KV_R, DH)
                s = lax.dot_general(
                    qv, ks, (((2,), (2,)), ((0,), (0,))),
                    preferred_element_type=jnp.float32)
                e = jnp.exp(s)
                o = lax.dot_general(
                    e.astype(jnp.bfloat16), vs, (((2,), (1,)), ((0,), (0,))),
                    preferred_element_type=jnp.float32)
                lsum = jnp.sum(e, axis=2)
                oa[:, hc] = o[:, :BQ, :].reshape(SQ, DH).astype(oa.dtype)
                ob[:, hc] = o[:, BQ:, :].reshape(SQ, DH).astype(ob.dtype)
                la[:, h:h + 1] = lsum[:, :BQ].reshape(SQ, 1)
                lb[:, h:h + 1] = lsum[:, BQ:].reshape(SQ, 1)

        qsend[2].wait_recv()
        qsend[1].wait_recv()
        partial2(3, 2, psrc.at[0], lsrc.at[0], psrc.at[1], lsrc.at[1])
        psend[0].start()
        lsend[0].start()
        psend[1].start()
        lsend[1].start()

        qsend[0].wait_recv()

        def _mid():
            psend2[0].start()

        partial2(1, 0, psrc.at[2], lsrc.at[2], pme_o, pme_l, mid=_mid)
        psend2[1].start()
        lsend[2].start()

        psend[0].wait_recv()
        lsend[0].wait_recv()
        psend[1].wait_recv()
        lsend[1].wait_recv()
        o01 = (pme_o[:, :] + pbuf[0, :, :].astype(jnp.float32)
               + pbuf[1, :, :].astype(jnp.float32))
        l01 = pme_l[:, :] + lbuf[0, :, :] + lbuf[1, :, :]
        wo16 = wo_ref[...].astype(jnp.bfloat16)
        lsend[2].wait_recv()
        l_all = l01 + lbuf[2, :, :]
        parts = []
        for i, (c0, c1, h0, h1) in enumerate(
                ((0, DH2, 0, HQ // 2), (DH2, D_MODEL, HQ // 2, HQ))):
            psend2[i].wait_recv()
            o_sum = o01[:, c0:c1] + pbuf[2, :, c0:c1].astype(jnp.float32)
            l_sum = l_all[:, h0:h1]
            ctx = jnp.concatenate(
                [o_sum[:, (h - h0) * DH:(h - h0 + 1) * DH]
                 / l_sum[:, h - h0:h - h0 + 1]
                 for h in range(h0, h1)], axis=1)
            parts.append(jnp.dot(ctx.astype(jnp.bfloat16), wo16[c0:c1, :],
                                 preferred_element_type=jnp.float32))
        out_ref[...] = parts[0] + parts[1]

        for k in range(3):
            qsend[k].wait_send()
            lsend[k].wait_send()
        for k in range(2):
            psend[k].wait_send()
            psend2[k].wait_send()

    out = pl.pallas_call(
        body,
        out_shape=jax.ShapeDtypeStruct((SQ, D_MODEL), jnp.float32),
        in_specs=[
            pl.BlockSpec(memory_space=pltpu.VMEM),
            pl.BlockSpec(memory_space=pltpu.VMEM),
            pl.BlockSpec(memory_space=pl.ANY),
            pl.BlockSpec(memory_space=pl.ANY),
            pl.BlockSpec(memory_space=pltpu.VMEM),
        ],
        out_specs=pl.BlockSpec(memory_space=pltpu.VMEM),
        scratch_shapes=[
            pltpu.VMEM((N_DEV, SQ, D_MODEL), jnp.bfloat16),
            pltpu.VMEM((3, SQ, D_MODEL), jnp.bfloat16),
            pltpu.VMEM((3, SQ, D_MODEL), jnp.bfloat16),
            pltpu.VMEM((3, SQ, HQ), jnp.float32),
            pltpu.VMEM((3, SQ, HQ), jnp.float32),
            pltpu.VMEM((SQ, D_MODEL), jnp.float32),
            pltpu.VMEM((SQ, HQ), jnp.float32),
            pltpu.VMEM((NB, GK, BQ, D_MODEL), jnp.bfloat16),
            pltpu.VMEM((NB, GK, BQ, D_MODEL), jnp.bfloat16),
            pltpu.SemaphoreType.DMA((3,)),
            pltpu.SemaphoreType.DMA((3,)),
            pltpu.SemaphoreType.DMA((4,)),
            pltpu.SemaphoreType.DMA((4,)),
            pltpu.SemaphoreType.DMA((3,)),
            pltpu.SemaphoreType.DMA((3,)),
            pltpu.SemaphoreType.DMA((2 * NB,)),
        ],
        compiler_params=pltpu.CompilerParams(
            collective_id=0, vmem_limit_bytes=60 * 1024 * 1024),
    )(x2, Wq, K4, V4, Wo)
    return out.reshape(1, SQ, D_MODEL)
